# baseline (speedup 1.0000x reference)
import functools

import jax
import jax.numpy as jnp
from jax import lax
from jax.experimental import pallas as pl
from jax.experimental.pallas import tpu as pltpu
from jax.experimental.pallas import tpu_sc as plsc

N_NODES = 10000
D = 128
N_PAD = 10240
STRIPE = N_PAD // 16
CHUNK = 128
N_EDGES = 320000
CHUNKS_PER_TILE = -(-N_EDGES // (32 * CHUNK))
E_TILE = CHUNKS_PER_TILE * CHUNK
E_PAD = E_TILE * 32
A_CUT = 53
B_CUT = 2 * CHUNKS_PER_TILE - A_CUT
MAXC = max(A_CUT, B_CUT)

_mesh = plsc.VectorSubcoreMesh(core_axis_name="c", subcore_axis_name="s")


@functools.partial(
    pl.kernel,
    out_type=jax.ShapeDtypeStruct((2, N_PAD, D), jnp.float32),
    mesh=_mesh,
    scratch_types=[
        pltpu.VMEM_SHARED((N_PAD, D), jnp.float32),
        pltpu.VMEM((CHUNKS_PER_TILE, CHUNK), jnp.int32),
        pltpu.VMEM((CHUNK, D), jnp.float32),
    ],
)
def _deg_kernel(dst_hbm, ones_hbm, zeros_hbm, out_hbm, deg_sh, dstv, onesv):
    c = lax.axis_index("c")
    s = lax.axis_index("s")
    row = pl.ds(s * STRIPE, STRIPE)
    pltpu.sync_copy(zeros_hbm.at[row], deg_sh.at[row])
    pltpu.sync_copy(dst_hbm.at[c, s], dstv)
    pltpu.sync_copy(ones_hbm, onesv)
    plsc.subcore_barrier()

    def body(j, carry):
        pltpu.sync_copy(onesv, deg_sh.at[dstv.at[j]], add=True)
        return carry

    lax.fori_loop(0, CHUNKS_PER_TILE, body, 0)
    plsc.subcore_barrier()
    pltpu.sync_copy(deg_sh.at[row], out_hbm.at[c, row])


@functools.partial(
    pl.kernel,
    out_type=jax.ShapeDtypeStruct((2, N_PAD, D), jnp.float32),
    mesh=_mesh,
    scratch_types=[
        pltpu.VMEM_SHARED((N_PAD, D), jnp.float32),
        pltpu.VMEM((MAXC, CHUNK), jnp.int32),
        pltpu.VMEM((MAXC, CHUNK), jnp.int32),
        pltpu.VMEM((CHUNK, D), jnp.float32),
        pltpu.SemaphoreType.DMA,
    ],
)
def _agg_kernel(g_hbm, src_hbm, dst_hbm, zeros_hbm, out_hbm,
                y_sh, srcv, dstv, rows, sem):
    c = lax.axis_index("c")
    s = lax.axis_index("s")
    row = pl.ds(s * STRIPE, STRIPE)

    @pl.when(c == 0)
    def _():
        pltpu.sync_copy(g_hbm.at[row], y_sh.at[row])

    @pl.when(c == 1)
    def _():
        pltpu.sync_copy(zeros_hbm.at[row], y_sh.at[row])

    pltpu.sync_copy(src_hbm.at[c, s], srcv)
    pltpu.sync_copy(dst_hbm.at[c, s], dstv)
    plsc.subcore_barrier()

    def body(j, carry):
        pltpu.async_copy(g_hbm.at[srcv.at[j]], rows, sem).wait()
        pltpu.sync_copy(rows, y_sh.at[dstv.at[j]], add=True)
        return carry

    lax.fori_loop(0, jnp.where(c == 0, A_CUT, B_CUT), body, 0)
    plsc.subcore_barrier()
    pltpu.sync_copy(y_sh.at[row], out_hbm.at[c, row])


def _dinv_block(d0_ref, d1_ref):
    deg = d0_ref[:, :1] + d1_ref[:, :1] + 1.0
    return lax.rsqrt(deg)


def _k1_body(x_ref, w_ref, d0_ref, d1_ref, g_ref):
    dinv = _dinv_block(d0_ref, d1_ref)
    g_ref[...] = dinv * jnp.dot(x_ref[...], w_ref[...],
                                preferred_element_type=jnp.float32)


def _k2_body(y_ref, d0_ref, d1_ref, b_ref, w_ref, g_ref):
    dinv = _dinv_block(d0_ref, d1_ref)
    h = jnp.maximum(dinv * (y_ref[0] + y_ref[1]) + b_ref[...], 0.0)
    g_ref[...] = dinv * jnp.dot(h, w_ref[...],
                                preferred_element_type=jnp.float32)


def _k3_body(y_ref, d0_ref, d1_ref, b_ref, o_ref):
    dinv = _dinv_block(d0_ref, d1_ref)
    o_ref[...] = dinv * (y_ref[0] + y_ref[1]) + b_ref[...]


_GRID = (N_PAD // STRIPE,)
_bs_rows = pl.BlockSpec((STRIPE, D), lambda i: (i, 0))
_bs_deg = pl.BlockSpec((STRIPE, D), lambda i: (i, 0))
_bs_w = pl.BlockSpec((D, D), lambda i: (0, 0))
_bs_b = pl.BlockSpec((1, D), lambda i: (0, 0))
_bs_y = pl.BlockSpec((2, STRIPE, D), lambda i: (0, i, 0))
_out_rows = jax.ShapeDtypeStruct((N_PAD, D), jnp.float32)

_k1 = pl.pallas_call(
    _k1_body, grid=_GRID,
    in_specs=[_bs_rows, _bs_w, _bs_deg, _bs_deg],
    out_specs=_bs_rows, out_shape=_out_rows)

_k2 = pl.pallas_call(
    _k2_body, grid=_GRID,
    in_specs=[_bs_y, _bs_deg, _bs_deg, _bs_b, _bs_w],
    out_specs=_bs_rows, out_shape=_out_rows)

_k3 = pl.pallas_call(
    _k3_body, grid=_GRID,
    in_specs=[_bs_y, _bs_deg, _bs_deg, _bs_b],
    out_specs=_bs_rows, out_shape=_out_rows)


def kernel(x, edge_index, W1, b1, W2, b2):
    src = edge_index[0].astype(jnp.int32)
    dst = edge_index[1].astype(jnp.int32)
    n_extra = E_PAD - N_EDGES
    pad_dst = N_NODES + (jnp.arange(n_extra, dtype=jnp.int32)
                         % (N_PAD - N_NODES))
    src_p = jnp.concatenate([src, jnp.zeros((n_extra,), jnp.int32)])
    dst_p = jnp.concatenate([dst, pad_dst])
    dst_r = dst_p.reshape(2, 16, CHUNKS_PER_TILE, CHUNK)

    def asym(flat, fill):
        split = 16 * A_CUT * CHUNK
        e0 = flat[:split].reshape(16, A_CUT, CHUNK)
        e0 = jnp.pad(e0, ((0, 0), (0, MAXC - A_CUT), (0, 0)),
                     constant_values=fill)
        e1 = flat[split:].reshape(16, B_CUT, CHUNK)
        e1 = jnp.pad(e1, ((0, 0), (0, MAXC - B_CUT), (0, 0)),
                     constant_values=fill)
        return jnp.stack([e0, e1])

    src_a = asym(src_p, 0)
    dst_a = asym(dst_p, N_NODES)

    x_pad = jnp.pad(x, ((0, N_PAD - N_NODES), (0, 0)))
    zeros128 = jnp.zeros((N_PAD, D), jnp.float32)
    ones_rows = jnp.ones((CHUNK, D), jnp.float32)
    b1r = b1.reshape(1, D)
    b2r = b2.reshape(1, D)

    deg_parts = _deg_kernel(dst_r, ones_rows, zeros128)
    d0, d1 = deg_parts[0], deg_parts[1]

    g1 = _k1(x_pad, W1, d0, d1)
    y1 = _agg_kernel(g1, src_a, dst_a, zeros128)
    g2 = _k2(y1, d0, d1, b1r, W2)
    y2 = _agg_kernel(g2, src_a, dst_a, zeros128)
    out_pad = _k3(y2, d0, d1, b2r)
    return out_pad[:N_NODES]


# speedup vs baseline: 1.2673x; 1.2673x over previous
import functools

import jax
import jax.numpy as jnp
from jax import lax
from jax.experimental import pallas as pl
from jax.experimental.pallas import tpu as pltpu
from jax.experimental.pallas import tpu_sc as plsc

N_NODES = 10000
D = 128
N_PAD = 10240
STRIPE = N_PAD // 16
CHUNK = 128
N_EDGES = 320000
CHUNKS_PER_TILE = -(-N_EDGES // (32 * CHUNK))
E_TILE = CHUNKS_PER_TILE * CHUNK
E_PAD = E_TILE * 32
A_CUT = 105
B_CUT = 2 * CHUNKS_PER_TILE - A_CUT
MAXC = max(A_CUT, B_CUT)

_mesh = plsc.VectorSubcoreMesh(core_axis_name="c", subcore_axis_name="s")


@functools.partial(
    pl.kernel,
    out_type=jax.ShapeDtypeStruct((2, N_PAD, D), jnp.float32),
    mesh=_mesh,
    scratch_types=[
        pltpu.VMEM_SHARED((N_PAD, D), jnp.float32),
        pltpu.VMEM((CHUNKS_PER_TILE, CHUNK), jnp.int32),
        pltpu.VMEM((CHUNK, D), jnp.float32),
    ],
)
def _deg_kernel(dst_hbm, ones_hbm, zeros_hbm, out_hbm, deg_sh, dstv, onesv):
    c = lax.axis_index("c")
    s = lax.axis_index("s")
    row = pl.ds(s * STRIPE, STRIPE)
    pltpu.sync_copy(zeros_hbm.at[row], deg_sh.at[row])
    pltpu.sync_copy(dst_hbm.at[c, s], dstv)
    pltpu.sync_copy(ones_hbm, onesv)
    plsc.subcore_barrier()

    def body(j, carry):
        pltpu.sync_copy(onesv, deg_sh.at[dstv.at[j]], add=True)
        return carry

    lax.fori_loop(0, CHUNKS_PER_TILE, body, 0)
    plsc.subcore_barrier()
    pltpu.sync_copy(deg_sh.at[row], out_hbm.at[c, row])


@functools.partial(
    pl.kernel,
    out_type=jax.ShapeDtypeStruct((2, N_PAD, D), jnp.float32),
    mesh=_mesh,
    scratch_types=[
        pltpu.VMEM_SHARED((N_PAD, D), jnp.float32),
        pltpu.VMEM((MAXC, CHUNK), jnp.int32),
        pltpu.VMEM((MAXC, CHUNK), jnp.int32),
        pltpu.VMEM((CHUNK, D), jnp.float32),
        pltpu.SemaphoreType.DMA,
    ],
)
def _agg_kernel(g_hbm, src_hbm, dst_hbm, zeros_hbm, out_hbm,
                y_sh, srcv, dstv, rows, sem):
    c = lax.axis_index("c")
    s = lax.axis_index("s")
    row = pl.ds(s * STRIPE, STRIPE)

    @pl.when(c == 0)
    def _():
        pltpu.sync_copy(g_hbm.at[row], y_sh.at[row])

    @pl.when(c == 1)
    def _():
        pltpu.sync_copy(zeros_hbm.at[row], y_sh.at[row])

    pltpu.sync_copy(src_hbm.at[c, s], srcv)
    pltpu.sync_copy(dst_hbm.at[c, s], dstv)
    plsc.subcore_barrier()

    def body(j, carry):
        pltpu.async_copy(g_hbm.at[srcv.at[j]], rows, sem).wait()
        pltpu.sync_copy(rows, y_sh.at[dstv.at[j]], add=True)
        return carry

    lax.fori_loop(0, jnp.where(c == 0, A_CUT, B_CUT), body, 0)
    plsc.subcore_barrier()
    pltpu.sync_copy(y_sh.at[row], out_hbm.at[c, row])


def _dinv_block(d0_ref, d1_ref):
    deg = d0_ref[:, :1] + d1_ref[:, :1] + 1.0
    return lax.rsqrt(deg)


def _k1_body(x_ref, w_ref, d0_ref, d1_ref, g_ref):
    dinv = _dinv_block(d0_ref, d1_ref)
    g_ref[...] = dinv * jnp.dot(x_ref[...], w_ref[...],
                                preferred_element_type=jnp.float32)


def _k2_body(y_ref, d0_ref, d1_ref, b_ref, w_ref, g_ref):
    dinv = _dinv_block(d0_ref, d1_ref)
    h = jnp.maximum(dinv * (y_ref[0] + y_ref[1]) + b_ref[...], 0.0)
    g_ref[...] = dinv * jnp.dot(h, w_ref[...],
                                preferred_element_type=jnp.float32)


def _k3_body(y_ref, d0_ref, d1_ref, b_ref, o_ref):
    dinv = _dinv_block(d0_ref, d1_ref)
    o_ref[...] = dinv * (y_ref[0] + y_ref[1]) + b_ref[...]


_GRID = (N_PAD // STRIPE,)
_bs_rows = pl.BlockSpec((STRIPE, D), lambda i: (i, 0))
_bs_deg = pl.BlockSpec((STRIPE, D), lambda i: (i, 0))
_bs_w = pl.BlockSpec((D, D), lambda i: (0, 0))
_bs_b = pl.BlockSpec((1, D), lambda i: (0, 0))
_bs_y = pl.BlockSpec((2, STRIPE, D), lambda i: (0, i, 0))
_out_rows = jax.ShapeDtypeStruct((N_PAD, D), jnp.float32)

_k1 = pl.pallas_call(
    _k1_body, grid=_GRID,
    in_specs=[_bs_rows, _bs_w, _bs_deg, _bs_deg],
    out_specs=_bs_rows, out_shape=_out_rows)

_k2 = pl.pallas_call(
    _k2_body, grid=_GRID,
    in_specs=[_bs_y, _bs_deg, _bs_deg, _bs_b, _bs_w],
    out_specs=_bs_rows, out_shape=_out_rows)

_k3 = pl.pallas_call(
    _k3_body, grid=_GRID,
    in_specs=[_bs_y, _bs_deg, _bs_deg, _bs_b],
    out_specs=_bs_rows, out_shape=_out_rows)


def kernel(x, edge_index, W1, b1, W2, b2):
    src = edge_index[0].astype(jnp.int32)
    dst = edge_index[1].astype(jnp.int32)
    n_extra = E_PAD - N_EDGES
    pad_dst = N_NODES + (jnp.arange(n_extra, dtype=jnp.int32)
                         % (N_PAD - N_NODES))
    src_p = jnp.concatenate([src, jnp.zeros((n_extra,), jnp.int32)])
    dst_p = jnp.concatenate([dst, pad_dst])
    dst_r = dst_p.reshape(2, 16, CHUNKS_PER_TILE, CHUNK)

    def asym(flat, fill):
        split = 16 * A_CUT * CHUNK
        e0 = flat[:split].reshape(16, A_CUT, CHUNK)
        e0 = jnp.pad(e0, ((0, 0), (0, MAXC - A_CUT), (0, 0)),
                     constant_values=fill)
        e1 = flat[split:].reshape(16, B_CUT, CHUNK)
        e1 = jnp.pad(e1, ((0, 0), (0, MAXC - B_CUT), (0, 0)),
                     constant_values=fill)
        return jnp.stack([e0, e1])

    src_a = asym(src_p, 0)
    dst_a = asym(dst_p, N_NODES)

    x_pad = jnp.pad(x, ((0, N_PAD - N_NODES), (0, 0)))
    zeros128 = jnp.zeros((N_PAD, D), jnp.float32)
    ones_rows = jnp.ones((CHUNK, D), jnp.float32)
    b1r = b1.reshape(1, D)
    b2r = b2.reshape(1, D)

    deg_parts = _deg_kernel(dst_r, ones_rows, zeros128)
    d0, d1 = deg_parts[0], deg_parts[1]

    g1 = _k1(x_pad, W1, d0, d1)
    y1 = _agg_kernel(g1, src_a, dst_a, zeros128)
    g2 = _k2(y1, d0, d1, b1r, W2)
    y2 = _agg_kernel(g2, src_a, dst_a, zeros128)
    out_pad = _k3(y2, d0, d1, b2r)
    return out_pad[:N_NODES]
